# Initial kernel scaffold; baseline (speedup 1.0000x reference)
import jax
import jax.numpy as jnp
from jax.experimental import pallas as pl


def kernel(x_seq, pbs_feat, rt_feat, nucl_table, pbs_table, rt_table):
    raise NotImplementedError("write your pallas kernel here")


# speedup vs baseline: 2.8361x; 2.8361x over previous
import functools
import jax
import jax.numpy as jnp
from jax.experimental import pallas as pl
from jax.experimental.pallas import tpu as pltpu

B, L = 4096, 200
NUCL_DIM, SPEC_DIM = 128, 32
OUT_DIM = NUCL_DIM + 2 * SPEC_DIM
BB = 16


def _body(x_ref, pbs_ref, rt_ref, nucl_ref, pbst_ref, rtt_ref, out_ref):
    x = x_ref[...]
    acc = jnp.zeros((BB, L, NUCL_DIM), jnp.float32)
    for k in range(5):
        sel = (x == k).astype(jnp.float32)[..., None]
        acc = acc + sel * nucl_ref[k][None, None, :]
    p = pbs_ref[...]
    r = rt_ref[...]
    pe = jnp.where(p > 0.5, pbst_ref[1][None, :], pbst_ref[0][None, :])
    re = jnp.where(r > 0.5, rtt_ref[1][None, :], rtt_ref[0][None, :])
    pe = jnp.broadcast_to(pe[:, None, :], (BB, L, SPEC_DIM))
    re = jnp.broadcast_to(re[:, None, :], (BB, L, SPEC_DIM))
    out_ref[...] = jnp.concatenate([acc, pe, re], axis=-1)


@jax.jit
def kernel(x_seq, pbs_feat, rt_feat, nucl_table, pbs_table, rt_table):
    pbs2 = pbs_feat.reshape(B, 1)
    rt2 = rt_feat.reshape(B, 1)
    grid = (B // BB,)
    return pl.pallas_call(
        _body,
        grid=grid,
        in_specs=[
            pl.BlockSpec((BB, L), lambda i: (i, 0)),
            pl.BlockSpec((BB, 1), lambda i: (i, 0)),
            pl.BlockSpec((BB, 1), lambda i: (i, 0)),
            pl.BlockSpec((5, NUCL_DIM), lambda i: (0, 0)),
            pl.BlockSpec((2, SPEC_DIM), lambda i: (0, 0)),
            pl.BlockSpec((2, SPEC_DIM), lambda i: (0, 0)),
        ],
        out_specs=pl.BlockSpec((BB, L, OUT_DIM), lambda i: (i, 0, 0)),
        out_shape=jax.ShapeDtypeStruct((B, L, OUT_DIM), jnp.float32),
        compiler_params=pltpu.CompilerParams(
            dimension_semantics=("arbitrary",),
        ),
    )(x_seq, pbs2, rt2, nucl_table, pbs_table, rt_table)
